# baseline (speedup 1.0000x reference)
import functools

import jax
import jax.numpy as jnp
from jax import lax
from jax.experimental import pallas as pl
from jax.experimental.pallas import tpu as pltpu
from jax.experimental.pallas import tpu_sc as plsc

POOL_SIZE = 100
PROMPT_LENGTH = 10
D_MODEL = 1024
TOP_K = 5
SEQ = 2048
PREFIX = TOP_K * PROMPT_LENGTH
RCHUNK = 256
NCHUNK = SEQ // RCHUNK
NB = 4


def _mean_body(x_ref, q_ref, acc_ref):
    r = pl.program_id(1)
    psum = jnp.sum(x_ref[0], axis=0, keepdims=True)

    @pl.when(r == 0)
    def _init():
        acc_ref[0:1, :] = psum

    @pl.when(r != 0)
    def _acc():
        acc_ref[0:1, :] += psum

    @pl.when(r == NCHUNK - 1)
    def _out():
        q_ref[0, 0:1, :] = acc_ref[0:1, :] * (1.0 / SEQ)


def _mean(x):
    return pl.pallas_call(
        _mean_body,
        grid=(NB, NCHUNK),
        in_specs=[pl.BlockSpec((1, RCHUNK, D_MODEL), lambda b, r: (b, r, 0))],
        out_specs=pl.BlockSpec((1, 8, D_MODEL), lambda b, r: (b, 0, 0)),
        out_shape=jax.ShapeDtypeStruct((NB, 8, D_MODEL), jnp.float32),
        scratch_shapes=[pltpu.VMEM((8, D_MODEL), jnp.float32)],
        compiler_params=pltpu.CompilerParams(
            dimension_semantics=("arbitrary", "arbitrary"),
        ),
    )(x)


SC_WIN = 32
SC_GATH = SC_WIN + 8
SC_NWIN = RCHUNK // SC_WIN


def _sc_copy_body(x_hbm, out_hbm, vbuf, sems):
    c = lax.axis_index("c")
    s = lax.axis_index("s")
    wid = s * 2 + c
    b = wid // NCHUNK
    r = lax.rem(wid, NCHUNK)
    base = r * RCHUNK

    for j in range(SC_NWIN):
        slot = j % 2

        if j >= 2:
            pltpu.make_async_copy(
                vbuf.at[slot, pl.ds(6, SC_WIN)],
                out_hbm.at[b, pl.ds(PREFIX, SC_WIN)],
                sems.at[slot],
            ).wait()

        first = (j == 0)
        if first:

            @pl.when(r == 0)
            def _edge():
                pltpu.sync_copy(
                    x_hbm.at[b, pl.ds(0, SC_WIN)],
                    vbuf.at[slot, pl.ds(8, SC_WIN)],
                )

            @pl.when(r != 0)
            def _normal():
                pltpu.sync_copy(
                    x_hbm.at[b, pl.ds(base - 8, SC_GATH)],
                    vbuf.at[slot, pl.ds(0, SC_GATH)],
                )
        else:
            pltpu.sync_copy(
                x_hbm.at[b, pl.ds(base + j * SC_WIN - 8, SC_GATH)],
                vbuf.at[slot, pl.ds(0, SC_GATH)],
            )

        pltpu.async_copy(
            vbuf.at[slot, pl.ds(6, SC_WIN)],
            out_hbm.at[b, pl.ds(PREFIX - 2 + base + j * SC_WIN, SC_WIN)],
            sems.at[slot],
        )

    for slot in range(2):
        pltpu.make_async_copy(
            vbuf.at[slot, pl.ds(6, SC_WIN)],
            out_hbm.at[b, pl.ds(PREFIX, SC_WIN)],
            sems.at[slot],
        ).wait()

    @pl.when(r == NCHUNK - 1)
    def _tail():
        pltpu.sync_copy(
            vbuf.at[(SC_NWIN - 1) % 2, pl.ds(38, 2)],
            out_hbm.at[b, pl.ds(PREFIX - 2 + SEQ, 2)],
        )


def _sc_copy(x):
    mesh = plsc.VectorSubcoreMesh(core_axis_name="c", subcore_axis_name="s")
    f = functools.partial(
        pl.kernel,
        out_type=jax.ShapeDtypeStruct((NB, PREFIX + SEQ, D_MODEL), jnp.float32),
        mesh=mesh,
        scratch_types=[
            pltpu.VMEM((2, SC_GATH, D_MODEL), jnp.float32),
            pltpu.SemaphoreType.DMA((2,)),
        ],
        compiler_params=pltpu.CompilerParams(use_tc_tiling_on_sc=False),
    )(_sc_copy_body)
    return f(x)


def _finish_body(out_in_ref, q_ref, pf_ref, keys_ref, out_ref, idx_ref, selbuf, sem):
    b = pl.program_id(0)
    q = q_ref[0, 0:1, :]
    qn = q / jnp.maximum(jnp.sqrt(jnp.sum(q * q)), 1e-12)

    k = keys_ref[:]
    knorm = jnp.sqrt(jnp.sum(k * k, axis=1, keepdims=True))
    kn = k / jnp.maximum(knorm, 1e-12)

    sim = lax.dot_general(
        qn, kn, (((1,), (1,)), ((), ())), preferred_element_type=jnp.float32
    )

    iota = lax.broadcasted_iota(jnp.int32, (1, POOL_SIZE), 1)
    idxs = []
    cur = sim
    for t in range(TOP_K):
        m = jnp.max(cur)
        it = jnp.min(jnp.where(cur == m, iota, POOL_SIZE))
        idx_ref[0, 0, t] = it
        idxs.append(it)
        cur = jnp.where(iota == it, -jnp.inf, cur)

    r_i = lax.broadcasted_iota(jnp.int32, (PREFIX, POOL_SIZE * PROMPT_LENGTH), 0)
    c_i = lax.broadcasted_iota(jnp.int32, (PREFIX, POOL_SIZE * PROMPT_LENGTH), 1)
    kk = r_i // PROMPT_LENGTH
    within = r_i % PROMPT_LENGTH
    sel_idx = jnp.zeros_like(kk)
    for t, it in enumerate(idxs):
        sel_idx = jnp.where(kk == t, it, sel_idx)
    oh = (c_i == sel_idx * PROMPT_LENGTH + within).astype(jnp.float32)
    selbuf[0:PREFIX, :] = lax.dot_general(
        oh, pf_ref[:], (((1,), (0,)), ((), ())),
        preferred_element_type=jnp.float32,
    )

    cp_a = pltpu.make_async_copy(
        selbuf.at[pl.ds(0, 48)], out_ref.at[b, pl.ds(0, 48)], sem
    )
    cp_b = pltpu.make_async_copy(
        selbuf.at[pl.ds(48, 2)], out_ref.at[b, pl.ds(48, 2)], sem
    )
    cp_a.start()
    cp_b.start()
    cp_a.wait()
    cp_b.wait()


def _finish(out_body, q8, pf, keys):
    return pl.pallas_call(
        _finish_body,
        grid=(NB,),
        in_specs=[
            pl.BlockSpec(memory_space=pl.ANY),
            pl.BlockSpec((1, 8, D_MODEL), lambda b: (b, 0, 0)),
            pl.BlockSpec((POOL_SIZE * PROMPT_LENGTH, D_MODEL), lambda b: (0, 0)),
            pl.BlockSpec((POOL_SIZE, D_MODEL), lambda b: (0, 0)),
        ],
        out_specs=[
            pl.BlockSpec(memory_space=pl.ANY),
            pl.BlockSpec(
                (1, 1, TOP_K), lambda b: (b, 0, 0), memory_space=pltpu.SMEM
            ),
        ],
        out_shape=[
            jax.ShapeDtypeStruct((NB, PREFIX + SEQ, D_MODEL), jnp.float32),
            jax.ShapeDtypeStruct((NB, 1, TOP_K), jnp.int32),
        ],
        scratch_shapes=[
            pltpu.VMEM((PREFIX + 6, D_MODEL), jnp.float32),
            pltpu.SemaphoreType.DMA,
        ],
        input_output_aliases={0: 0},
    )(out_body, q8, pf, keys)


@functools.partial(jax.jit)
def kernel(x, prompts, keys):
    B = x.shape[0]
    pf = prompts.reshape(POOL_SIZE * PROMPT_LENGTH, D_MODEL)
    q8 = _mean(x)
    out_body = _sc_copy(x)
    out, idx3 = _finish(out_body, q8, pf, keys)
    return (out, idx3.reshape(B, TOP_K))


# speedup vs baseline: 2.2811x; 2.2811x over previous
import functools

import jax
import jax.numpy as jnp
from jax import lax
from jax.experimental import pallas as pl
from jax.experimental.pallas import tpu as pltpu

POOL_SIZE = 100
PROMPT_LENGTH = 10
D_MODEL = 1024
TOP_K = 5
SEQ = 2048
PREFIX = TOP_K * PROMPT_LENGTH


def _body(x_ref, p_ref, keys_ref, out_ref, idx_ref):
    s = x_ref[0]
    q = jnp.sum(s, axis=0, keepdims=True) * (1.0 / SEQ)
    qn = q / jnp.maximum(jnp.sqrt(jnp.sum(q * q)), 1e-12)

    k = keys_ref[:]
    knorm = jnp.sqrt(jnp.sum(k * k, axis=1, keepdims=True))
    kn = k / jnp.maximum(knorm, 1e-12)

    sim = lax.dot_general(
        qn, kn, (((1,), (1,)), ((), ())), preferred_element_type=jnp.float32
    )

    iota = lax.broadcasted_iota(jnp.int32, (1, POOL_SIZE), 1)
    cur = sim
    for t in range(TOP_K):
        m = jnp.max(cur)
        it = jnp.min(jnp.where(cur == m, iota, POOL_SIZE))
        idx_ref[0, 0, t] = it
        cur = jnp.where(iota == it, -jnp.inf, cur)
        out_ref[0, t * PROMPT_LENGTH : (t + 1) * PROMPT_LENGTH, :] = p_ref[it]

    out_ref[0, PREFIX:, :] = s


@functools.partial(jax.jit)
def kernel(x, prompts, keys):
    B = x.shape[0]
    out, idx3 = pl.pallas_call(
        _body,
        grid=(B,),
        in_specs=[
            pl.BlockSpec((1, SEQ, D_MODEL), lambda b: (b, 0, 0)),
            pl.BlockSpec(
                (POOL_SIZE, PROMPT_LENGTH, D_MODEL), lambda b: (0, 0, 0)
            ),
            pl.BlockSpec((POOL_SIZE, D_MODEL), lambda b: (0, 0)),
        ],
        out_specs=[
            pl.BlockSpec((1, PREFIX + SEQ, D_MODEL), lambda b: (b, 0, 0)),
            pl.BlockSpec((1, 1, TOP_K), lambda b: (b, 0, 0), memory_space=pltpu.SMEM),
        ],
        out_shape=[
            jax.ShapeDtypeStruct((B, PREFIX + SEQ, D_MODEL), jnp.float32),
            jax.ShapeDtypeStruct((B, 1, TOP_K), jnp.int32),
        ],
    )(x, prompts, keys)
    return (out, idx3.reshape(B, TOP_K))
